# Initial kernel scaffold; baseline (speedup 1.0000x reference)
import functools

import jax
import jax.numpy as jnp
from jax import lax
from jax.experimental import pallas as pl
from jax.experimental.pallas import tpu as pltpu
from jax.experimental.pallas import tpu_sc as plsc

N = 10000
E = 320000
D = 128
G = 256
C = 40
EPS = 1e-5

CHUNK = 128
CH_PER_TILE = 79
NW = 32
CH_TOTAL = CH_PER_TILE * NW
E_PAD = CH_TOTAL * CHUNK
ACC = 10240
SEG = ACC // 16

_mesh = plsc.VectorSubcoreMesh(core_axis_name="c", subcore_axis_name="s")


@functools.partial(
    pl.kernel,
    out_type=jax.ShapeDtypeStruct((NW, ACC), jnp.float32),
    mesh=_mesh,
    scratch_types=[
        pltpu.VMEM((1, CHUNK), jnp.int32),
        pltpu.VMEM((ACC,), jnp.float32),
    ],
)
def _sc_count(row_hbm, out_hbm, idx_v, acc_v):
    c = lax.axis_index("c")
    s = lax.axis_index("s")
    w = c * 16 + s
    zero16 = jnp.zeros((16,), jnp.float32)
    ones16 = jnp.ones((16,), jnp.float32)

    @pl.loop(0, ACC, step=16)
    def _(i):
        acc_v[pl.ds(i, 16)] = zero16

    @pl.loop(0, CH_PER_TILE)
    def _(it):
        j = w * CH_PER_TILE + it
        pltpu.sync_copy(row_hbm.at[pl.ds(j, 1)], idx_v)

        @pl.loop(0, CHUNK, step=16)
        def _(k):
            iv = idx_v[0, pl.ds(k, 16)]
            plsc.addupdate_scatter(acc_v, [iv], ones16)

    pltpu.sync_copy(acc_v, out_hbm.at[w])


@functools.partial(
    pl.kernel,
    out_type=jax.ShapeDtypeStruct((2, ACC, D), jnp.float32),
    mesh=_mesh,
    scratch_types=[
        pltpu.VMEM((1, CHUNK), jnp.int32),
        pltpu.VMEM((1, CHUNK), jnp.int32),
        pltpu.VMEM((CHUNK, D), jnp.float32),
        pltpu.VMEM_SHARED((ACC, D), jnp.float32),
        pltpu.SemaphoreType.DMA,
    ],
)
def _sc_mp(h_hbm, rowg_hbm, cols_hbm, out_hbm, ridx, cidx, rows_v, acc_sh, sem):
    c = lax.axis_index("c")
    s = lax.axis_index("s")
    zero16 = jnp.zeros((16,), jnp.float32)

    @pl.loop(0, CHUNK)
    def _(r):
        @pl.loop(0, D, step=16)
        def _(l):
            rows_v[r, pl.ds(l, 16)] = zero16

    @pl.loop(0, SEG, step=CHUNK)
    def _(i):
        pltpu.sync_copy(rows_v, acc_sh.at[pl.ds(s * SEG + i, CHUNK)])

    plsc.subcore_barrier()

    base = (c * 16 + s) * CH_PER_TILE

    @pl.loop(0, CH_PER_TILE)
    def _(it):
        j = base + it
        pltpu.sync_copy(rowg_hbm.at[pl.ds(j, 1)], ridx)
        pltpu.sync_copy(cols_hbm.at[pl.ds(j, 1)], cidx)
        pltpu.async_copy(h_hbm.at[ridx.at[0]], rows_v, sem).wait()
        pltpu.sync_copy(rows_v, acc_sh.at[cidx.at[0]], add=True)

    plsc.subcore_barrier()
    pltpu.sync_copy(acc_sh.at[pl.ds(s * SEG, SEG)],
                    out_hbm.at[c, pl.ds(s * SEG, SEG)])


def _tc1_body(cnt_ref, x_ref, w1_ref, dis_ref, h_ref):
    cnt = cnt_ref[...]
    ones = jnp.ones((NW, 1), jnp.float32)
    deg = lax.dot_general(cnt, ones, (((0,), (0,)), ((), ())))
    dis = lax.rsqrt(deg + 1.0)
    dis_ref[...] = dis
    h = lax.dot_general(x_ref[...], w1_ref[...], (((1,), (1,)), ((), ())))
    h_ref[...] = h * dis[:N]


_tc1 = pl.pallas_call(
    _tc1_body,
    out_shape=(jax.ShapeDtypeStruct((ACC, 1), jnp.float32),
               jax.ShapeDtypeStruct((N, D), jnp.float32)),
)


def _tc2_body(part_ref, dis_ref, g_ref, b_ref, w_ref, out_ref):
    dis = dis_ref[...][:N]
    sarr = (part_ref[0, :N, :] + part_ref[1, :N, :]) * dis
    m = jnp.mean(sarr, axis=0, keepdims=True)
    d = sarr - m
    v = jnp.mean(d * d, axis=0, keepdims=True)
    h = d * lax.rsqrt(v + EPS) * g_ref[...] + b_ref[...]
    h = jnp.maximum(h, 0.0)
    out_ref[...] = lax.dot_general(
        h, w_ref[...], (((1,), (1,)), ((), ()))) * dis


_tc2 = pl.pallas_call(
    _tc2_body,
    out_shape=jax.ShapeDtypeStruct((N, D), jnp.float32),
)


def _tc3_body(part_ref, dis_ref, bat_ref, fw1_ref, fb1_ref, fw2_ref, fb2_ref,
              out_ref):
    dis = dis_ref[...][:N]
    h = (part_ref[0, :N, :] + part_ref[1, :N, :]) * dis
    gid = lax.broadcasted_iota(jnp.int32, (G, N), 0)
    oh = (gid == bat_ref[...]).astype(jnp.float32)
    pooled = lax.dot_general(oh, h, (((1,), (0,)), ((), ())))
    cnt = jnp.sum(oh, axis=1, keepdims=True)
    xg = pooled / jnp.maximum(cnt, 1.0)
    a = lax.dot_general(xg, fw1_ref[...], (((1,), (1,)), ((), ()))) + fb1_ref[...]
    a = jnp.maximum(a, 0.0)
    z = lax.dot_general(a, fw2_ref[...], (((1,), (1,)), ((), ()))) + fb2_ref[...]
    zmax = jnp.max(z, axis=1, keepdims=True)
    lse = zmax + jnp.log(jnp.sum(jnp.exp(z - zmax), axis=1, keepdims=True))
    out_ref[...] = z - lse


_tc3 = pl.pallas_call(
    _tc3_body,
    out_shape=jax.ShapeDtypeStruct((G, C), jnp.float32),
)


def kernel(x, edge_index, batch, W1, W2, W3, g1, beta1, g2, beta2,
           fW1, fb1, fW2, fb2):
    row = edge_index[0].astype(jnp.int32)
    col = edge_index[1].astype(jnp.int32)
    npad = E_PAD - E
    rowg = jnp.concatenate([row, jnp.zeros((npad,), jnp.int32)])
    rowc = jnp.concatenate([row, jnp.full((npad,), N, jnp.int32)])
    cols = jnp.concatenate([col, jnp.full((npad,), N, jnp.int32)])
    rowg = rowg.reshape(CH_TOTAL, CHUNK)
    rowc = rowc.reshape(CH_TOTAL, CHUNK)
    cols = cols.reshape(CH_TOTAL, CHUNK)
    bat = batch.astype(jnp.int32).reshape(1, N)

    cnt = _sc_count(rowc)
    dis, h1 = _tc1(cnt, x, W1)
    p1 = _sc_mp(h1, rowg, cols)
    h2 = _tc2(p1, dis, g1.reshape(1, D), beta1.reshape(1, D), W2)
    p2 = _sc_mp(h2, rowg, cols)
    h3 = _tc2(p2, dis, g2.reshape(1, D), beta2.reshape(1, D), W3)
    p3 = _sc_mp(h3, rowg, cols)
    return _tc3(p3, dis, bat, fW1, fb1.reshape(1, C), fW2, fb2.reshape(1, C))


# speedup vs baseline: 7.6455x; 7.6455x over previous
import functools

import jax
import jax.numpy as jnp
from jax import lax
from jax.experimental import pallas as pl
from jax.experimental.pallas import tpu as pltpu
from jax.experimental.pallas import tpu_sc as plsc

N = 10000
E = 320000
D = 128
G = 256
C = 40
EPS = 1e-5

CHUNK = 128
CH_PER_TILE = 79
NW = 32
CH_TOTAL = CH_PER_TILE * NW
E_PAD = CH_TOTAL * CHUNK
ACC = 10240
SEG = ACC // 16

_mesh = plsc.VectorSubcoreMesh(core_axis_name="c", subcore_axis_name="s")

_sc_params = pltpu.CompilerParams()
if "needs_layout_passes" in pltpu.CompilerParams.__dataclass_fields__:
    import dataclasses as _dc
    _sc_params = _dc.replace(_sc_params, needs_layout_passes=False)


@functools.partial(
    pl.kernel,
    out_type=jax.ShapeDtypeStruct((NW, ACC), jnp.float32),
    mesh=_mesh,
    scratch_types=[
        pltpu.VMEM((1, CHUNK), jnp.int32),
        pltpu.VMEM((ACC,), jnp.float32),
    ],
    compiler_params=_sc_params,
)
def _sc_count(row_hbm, out_hbm, idx_v, acc_v):
    c = lax.axis_index("c")
    s = lax.axis_index("s")
    w = c * 16 + s
    zero16 = jnp.zeros((16,), jnp.float32)
    ones16 = jnp.ones((16,), jnp.float32)

    @pl.loop(0, ACC, step=16)
    def _(i):
        acc_v[pl.ds(i, 16)] = zero16

    @pl.loop(0, CH_PER_TILE)
    def _(it):
        j = w * CH_PER_TILE + it
        pltpu.sync_copy(row_hbm.at[pl.ds(j, 1)], idx_v)

        @pl.loop(0, CHUNK, step=16)
        def _(k):
            iv = idx_v[0, pl.ds(k, 16)]
            plsc.addupdate_scatter(acc_v, [iv], ones16)

    pltpu.sync_copy(acc_v, out_hbm.at[w])


@functools.partial(
    pl.kernel,
    out_type=jax.ShapeDtypeStruct((2, ACC, D), jnp.float32),
    mesh=_mesh,
    scratch_types=[
        pltpu.VMEM((1, CHUNK), jnp.int32),
        pltpu.VMEM((1, CHUNK), jnp.int32),
        pltpu.VMEM((CHUNK, D), jnp.float32),
        pltpu.VMEM_SHARED((ACC, D), jnp.float32),
        pltpu.SemaphoreType.DMA,
    ],
    compiler_params=_sc_params,
)
def _sc_mp(h_hbm, rowg_hbm, cols_hbm, out_hbm, ridx, cidx, rows_v, acc_sh, sem):
    c = lax.axis_index("c")
    s = lax.axis_index("s")
    zero16 = jnp.zeros((16,), jnp.float32)

    @pl.loop(0, CHUNK)
    def _(r):
        @pl.loop(0, D, step=16)
        def _(l):
            rows_v[r, pl.ds(l, 16)] = zero16

    @pl.loop(0, SEG, step=CHUNK)
    def _(i):
        pltpu.sync_copy(rows_v, acc_sh.at[pl.ds(s * SEG + i, CHUNK)])

    plsc.subcore_barrier()

    base = (c * 16 + s) * CH_PER_TILE

    @pl.loop(0, CH_PER_TILE)
    def _(it):
        j = base + it
        pltpu.sync_copy(rowg_hbm.at[pl.ds(j, 1)], ridx)
        pltpu.sync_copy(cols_hbm.at[pl.ds(j, 1)], cidx)
        pltpu.async_copy(h_hbm.at[ridx.at[0]], rows_v, sem).wait()
        pltpu.sync_copy(rows_v, acc_sh.at[cidx.at[0]], add=True)

    plsc.subcore_barrier()
    pltpu.sync_copy(acc_sh.at[pl.ds(s * SEG, SEG)],
                    out_hbm.at[c, pl.ds(s * SEG, SEG)])


def _tc1_body(cnt_ref, x_ref, w1_ref, dis_ref, h_ref):
    cnt = cnt_ref[...]
    ones = jnp.ones((NW, 1), jnp.float32)
    deg = lax.dot_general(cnt, ones, (((0,), (0,)), ((), ())))
    dis = lax.rsqrt(deg + 1.0)
    dis_ref[...] = dis
    h = lax.dot_general(x_ref[...], w1_ref[...], (((1,), (1,)), ((), ())))
    h_ref[...] = h * dis[:N]


_tc1 = pl.pallas_call(
    _tc1_body,
    out_shape=(jax.ShapeDtypeStruct((ACC, 1), jnp.float32),
               jax.ShapeDtypeStruct((N, D), jnp.float32)),
)


def _tc2_body(part_ref, dis_ref, g_ref, b_ref, w_ref, out_ref):
    dis = dis_ref[...][:N]
    sarr = (part_ref[0, :N, :] + part_ref[1, :N, :]) * dis
    m = jnp.mean(sarr, axis=0, keepdims=True)
    d = sarr - m
    v = jnp.mean(d * d, axis=0, keepdims=True)
    h = d * lax.rsqrt(v + EPS) * g_ref[...] + b_ref[...]
    h = jnp.maximum(h, 0.0)
    out_ref[...] = lax.dot_general(
        h, w_ref[...], (((1,), (1,)), ((), ()))) * dis


_tc2 = pl.pallas_call(
    _tc2_body,
    out_shape=jax.ShapeDtypeStruct((N, D), jnp.float32),
)


def _tc3_body(part_ref, dis_ref, bat_ref, fw1_ref, fb1_ref, fw2_ref, fb2_ref,
              out_ref):
    dis = dis_ref[...][:N]
    h = (part_ref[0, :N, :] + part_ref[1, :N, :]) * dis
    gid = lax.broadcasted_iota(jnp.int32, (G, N), 0)
    oh = (gid == bat_ref[...]).astype(jnp.float32)
    pooled = lax.dot_general(oh, h, (((1,), (0,)), ((), ())))
    cnt = jnp.sum(oh, axis=1, keepdims=True)
    xg = pooled / jnp.maximum(cnt, 1.0)
    a = lax.dot_general(xg, fw1_ref[...], (((1,), (1,)), ((), ()))) + fb1_ref[...]
    a = jnp.maximum(a, 0.0)
    z = lax.dot_general(a, fw2_ref[...], (((1,), (1,)), ((), ()))) + fb2_ref[...]
    zmax = jnp.max(z, axis=1, keepdims=True)
    lse = zmax + jnp.log(jnp.sum(jnp.exp(z - zmax), axis=1, keepdims=True))
    out_ref[...] = z - lse


_tc3 = pl.pallas_call(
    _tc3_body,
    out_shape=jax.ShapeDtypeStruct((G, C), jnp.float32),
)


def kernel(x, edge_index, batch, W1, W2, W3, g1, beta1, g2, beta2,
           fW1, fb1, fW2, fb2):
    row = edge_index[0].astype(jnp.int32)
    col = edge_index[1].astype(jnp.int32)
    npad = E_PAD - E
    rowg = jnp.concatenate([row, jnp.zeros((npad,), jnp.int32)])
    rowc = jnp.concatenate([row, jnp.full((npad,), N, jnp.int32)])
    cols = jnp.concatenate([col, jnp.full((npad,), N, jnp.int32)])
    rowg = rowg.reshape(CH_TOTAL, CHUNK)
    rowc = rowc.reshape(CH_TOTAL, CHUNK)
    cols = cols.reshape(CH_TOTAL, CHUNK)
    bat = batch.astype(jnp.int32).reshape(1, N)

    cnt = _sc_count(rowc)
    dis, h1 = _tc1(cnt, x, W1)
    p1 = _sc_mp(h1, rowg, cols)
    h2 = _tc2(p1, dis, g1.reshape(1, D), beta1.reshape(1, D), W2)
    p2 = _sc_mp(h2, rowg, cols)
    h3 = _tc2(p2, dis, g2.reshape(1, D), beta2.reshape(1, D), W3)
    p3 = _sc_mp(h3, rowg, cols)
    return _tc3(p3, dis, bat, fW1, fb1.reshape(1, D), fW2, fb2.reshape(1, C))
